# Initial kernel scaffold; baseline (speedup 1.0000x reference)
import jax
import jax.numpy as jnp
from jax.experimental import pallas as pl


def kernel(grad_output_batch, sign_ids, similarities, edge_src, edge_dst, edge_weight, num_diseases):
    raise NotImplementedError("write your pallas kernel here")


# speedup vs baseline: 3.0412x; 3.0412x over previous
import functools

import jax
import jax.numpy as jnp
from jax import lax
from jax.experimental import pallas as pl
from jax.experimental.pallas import tpu as pltpu
from jax.experimental.pallas import tpu_sc as plsc

ETA = 0.01
B, F, S, NS, ND = 4096, 2048, 64, 4096, 128
NC, NSUB, LANES = 2, 16, 16
NGRP = ND // 16
GRP_WORDS = NS * 16
PAD_BASE = 4 * GRP_WORDS
SP_WORDS = PAD_BASE + 2048
ZCHUNK = SP_WORDS // NSUB

EPT = 3328
EPAD = EPT * NSUB
NCH = EPT // 128


def _norm_body(x_ref, o_ref):
    x = x_ref[...]
    o_ref[...] = ETA * jnp.sqrt(jnp.sum(x * x, axis=1, keepdims=True))


_norm_call = pl.pallas_call(
    _norm_body,
    grid=(8,),
    in_specs=[pl.BlockSpec((B // 8, F), lambda i: (i, 0))],
    out_specs=pl.BlockSpec((B // 8, 1), lambda i: (i, 0)),
    out_shape=jax.ShapeDtypeStruct((B, 1), jnp.float32),
)


def _abuild_body(src_hbm, dst_hbm, w_hbm, nd_hbm, out_hbm,
                 src_v, dst_v, w_v, nd_v, idx_v, upd_v, zbuf, a_sp):
    c = lax.axis_index("c")
    s = lax.axis_index("s")

    zero = jnp.zeros((16,), jnp.float32)

    def _zb(i, carry):
        zbuf[pl.ds(i * 16, 16)] = zero
        return carry

    lax.fori_loop(0, ZCHUNK // 16, _zb, 0)
    pltpu.sync_copy(zbuf, a_sp.at[pl.ds(s * ZCHUNK, ZCHUNK)])
    plsc.subcore_barrier()

    base = s * EPT
    pltpu.sync_copy(src_hbm.at[pl.ds(base, EPT)], src_v)
    pltpu.sync_copy(dst_hbm.at[pl.ds(base, EPT)], dst_v)
    pltpu.sync_copy(w_hbm.at[pl.ds(base, EPT)], w_v)
    pltpu.sync_copy(nd_hbm, nd_v)

    lo = c * (ND // NC)
    ndv = nd_v[...]
    iota = lax.iota(jnp.int32, 16)
    pad_slots = PAD_BASE + iota * 16 + s

    def _chunk(r, carry):
        for k in range(8):
            off = r * 128 + k * 16
            sv = src_v[pl.ds(off, 16)]
            dv = dst_v[pl.ds(off, 16)]
            wv = w_v[pl.ds(off, 16)]
            local = sv - lo
            ok = (sv >= lo) & (sv < lo + (ND // NC)) & (sv < ndv)
            flat = (local >> 4) * GRP_WORDS + dv * 16 + (local & 15)
            idx_v[r, pl.ds(k * 16, 16)] = jnp.where(ok, flat, pad_slots)
            upd_v[r, pl.ds(k * 16, 16)] = jnp.where(ok, wv, 0.0)
        pltpu.sync_copy(upd_v.at[r], a_sp.at[idx_v.at[r]], add=True)
        return carry

    lax.fori_loop(0, NCH, _chunk, 0)
    plsc.subcore_barrier()

    @pl.when(s < 4)
    def _():
        pltpu.sync_copy(a_sp.at[pl.ds(s * GRP_WORDS, GRP_WORDS)],
                        out_hbm.at[c * 4 + s])


@functools.cache
def _abuild_call():
    return pl.kernel(
        _abuild_body,
        out_type=jax.ShapeDtypeStruct((NGRP, GRP_WORDS), jnp.float32),
        mesh=plsc.VectorSubcoreMesh(core_axis_name="c", subcore_axis_name="s",
                                    num_cores=NC, num_subcores=NSUB),
        compiler_params=pltpu.CompilerParams(needs_layout_passes=False),
        scratch_types=[
            pltpu.VMEM((EPT,), jnp.int32),
            pltpu.VMEM((EPT,), jnp.int32),
            pltpu.VMEM((EPT,), jnp.float32),
            pltpu.VMEM((16,), jnp.int32),
            pltpu.VMEM((NCH, 128), jnp.int32),
            pltpu.VMEM((NCH, 128), jnp.float32),
            pltpu.VMEM((ZCHUNK,), jnp.float32),
            pltpu.VMEM_SHARED((SP_WORDS,), jnp.float32),
        ],
    )


def _gmm_body(a_hbm, ids_hbm, sims_hbm, g_hbm, out_hbm,
              a_v, ids_v, sims_v, g_v, out_v):
    c = lax.axis_index("c")
    s = lax.axis_index("s")
    wid = s * NC + c
    grp = wid % NGRP
    sb = wid // NGRP

    pltpu.sync_copy(a_hbm.at[grp], a_v)
    cvecs = [jnp.full((16,), cc, jnp.int32) for cc in range(16)]

    def _chunk(k, carry):
        off = sb * 1024 + k * 256
        pltpu.sync_copy(ids_hbm.at[:, pl.ds(off, 256)], ids_v)
        pltpu.sync_copy(sims_hbm.at[:, pl.ds(off, 256)], sims_v)
        pltpu.sync_copy(g_hbm.at[pl.ds(off, 256)], g_v)

        def _group(b, carry2):
            col0 = b * 16

            def _jstep(j, acc):
                sids = ids_v[j, pl.ds(col0, 16)]
                simv = sims_v[j, pl.ds(col0, 16)]
                base = sids << 4
                return tuple(
                    acc[cc] + simv * plsc.load_gather(a_v, [base + cvecs[cc]])
                    for cc in range(16))

            acc0 = tuple(jnp.zeros((16,), jnp.float32) for _ in range(16))
            acc = lax.fori_loop(0, S, _jstep, acc0)
            scale = g_v[pl.ds(col0, 16)]
            for cc in range(16):
                out_v[cc, pl.ds(col0, 16)] = acc[cc] * scale
            return carry2

        lax.fori_loop(0, 16, _group, 0)
        pltpu.sync_copy(out_v, out_hbm.at[pl.ds(grp * 16, 16),
                                          pl.ds(off, 256)])
        return carry

    lax.fori_loop(0, 4, _chunk, 0)


@functools.cache
def _gmm_call():
    return pl.kernel(
        _gmm_body,
        out_type=jax.ShapeDtypeStruct((ND, B), jnp.float32),
        mesh=plsc.VectorSubcoreMesh(core_axis_name="c", subcore_axis_name="s",
                                    num_cores=NC, num_subcores=NSUB),
        compiler_params=pltpu.CompilerParams(needs_layout_passes=False),
        scratch_types=[
            pltpu.VMEM((NS * 16,), jnp.float32),
            pltpu.VMEM((S, 256), jnp.int32),
            pltpu.VMEM((S, 256), jnp.float32),
            pltpu.VMEM((256,), jnp.float32),
            pltpu.VMEM((16, 256), jnp.float32),
        ],
    )


def kernel(grad_output_batch, sign_ids, similarities, edge_src, edge_dst,
           edge_weight, num_diseases):
    eta_g = _norm_call(grad_output_batch).reshape(B)

    src = edge_src.astype(jnp.int32)
    dst = edge_dst.astype(jnp.int32)
    w = edge_weight.astype(jnp.float32)
    npad = EPAD - src.shape[0]
    src = jnp.concatenate([src, jnp.zeros((npad,), jnp.int32)])
    dst = jnp.concatenate([dst, jnp.zeros((npad,), jnp.int32)])
    w = jnp.concatenate([w, jnp.zeros((npad,), jnp.float32)])
    nd_arr = jnp.full((16,), num_diseases, jnp.int32)

    a_grouped = _abuild_call()(src, dst, w, nd_arr)

    ids_t = sign_ids.astype(jnp.int32).T
    sims_t = similarities.T
    out_t = _gmm_call()(a_grouped, ids_t, sims_t, eta_g)
    return out_t.T


# speedup vs baseline: 4.5590x; 1.4991x over previous
import functools

import jax
import jax.numpy as jnp
from jax import lax
from jax.experimental import pallas as pl
from jax.experimental.pallas import tpu as pltpu
from jax.experimental.pallas import tpu_sc as plsc

ETA = 0.01
B, F, S, NS, ND = 4096, 2048, 64, 4096, 128
NC, NSUB, LANES = 2, 16, 16
NGRP = ND // 16
GRP_WORDS = NS * 16
PAD_BASE = 4 * GRP_WORDS
SP_WORDS = PAD_BASE + 2048
ZCHUNK = SP_WORDS // NSUB

EPT = 3328
EPAD = EPT * NSUB
NCH = EPT // 128


def _norm_body(x_ref, o_ref):
    x = x_ref[...]
    o_ref[...] = ETA * jnp.sqrt(jnp.sum(x * x, axis=1, keepdims=True))


_norm_call = pl.pallas_call(
    _norm_body,
    grid=(8,),
    in_specs=[pl.BlockSpec((B // 8, F), lambda i: (i, 0))],
    out_specs=pl.BlockSpec((B // 8, 1), lambda i: (i, 0)),
    out_shape=jax.ShapeDtypeStruct((B, 1), jnp.float32),
)


def _abuild_body(src_hbm, dst_hbm, w_hbm, nd_hbm, out_hbm,
                 src_v, dst_v, w_v, nd_v, idx_v, upd_v, zbuf, a_sp):
    c = lax.axis_index("c")
    s = lax.axis_index("s")

    zero = jnp.zeros((16,), jnp.float32)

    def _zb(i, carry):
        zbuf[pl.ds(i * 16, 16)] = zero
        return carry

    lax.fori_loop(0, ZCHUNK // 16, _zb, 0)
    pltpu.sync_copy(zbuf, a_sp.at[pl.ds(s * ZCHUNK, ZCHUNK)])
    plsc.subcore_barrier()

    base = s * EPT
    pltpu.sync_copy(src_hbm.at[pl.ds(base, EPT)], src_v)
    pltpu.sync_copy(dst_hbm.at[pl.ds(base, EPT)], dst_v)
    pltpu.sync_copy(w_hbm.at[pl.ds(base, EPT)], w_v)
    pltpu.sync_copy(nd_hbm, nd_v)

    lo = c * (ND // NC)
    ndv = nd_v[...]
    iota = lax.iota(jnp.int32, 16)
    pad_slots = PAD_BASE + iota * 16 + s

    def _chunk(r, carry):
        for k in range(8):
            off = r * 128 + k * 16
            sv = src_v[pl.ds(off, 16)]
            dv = dst_v[pl.ds(off, 16)]
            wv = w_v[pl.ds(off, 16)]
            local = sv - lo
            ok = (sv >= lo) & (sv < lo + (ND // NC)) & (sv < ndv)
            flat = (local >> 4) * GRP_WORDS + dv * 16 + (local & 15)
            idx_v[r, pl.ds(k * 16, 16)] = jnp.where(ok, flat, pad_slots)
            upd_v[r, pl.ds(k * 16, 16)] = jnp.where(ok, wv, 0.0)
        pltpu.sync_copy(upd_v.at[r], a_sp.at[idx_v.at[r]], add=True)
        return carry

    lax.fori_loop(0, NCH, _chunk, 0)
    plsc.subcore_barrier()

    @pl.when(s < 4)
    def _():
        pltpu.sync_copy(a_sp.at[pl.ds(s * GRP_WORDS, GRP_WORDS)],
                        out_hbm.at[c * 4 + s])


@functools.cache
def _abuild_call():
    return pl.kernel(
        _abuild_body,
        out_type=jax.ShapeDtypeStruct((NGRP, GRP_WORDS), jnp.float32),
        mesh=plsc.VectorSubcoreMesh(core_axis_name="c", subcore_axis_name="s",
                                    num_cores=NC, num_subcores=NSUB),
        compiler_params=pltpu.CompilerParams(needs_layout_passes=False),
        scratch_types=[
            pltpu.VMEM((EPT,), jnp.int32),
            pltpu.VMEM((EPT,), jnp.int32),
            pltpu.VMEM((EPT,), jnp.float32),
            pltpu.VMEM((16,), jnp.int32),
            pltpu.VMEM((NCH, 128), jnp.int32),
            pltpu.VMEM((NCH, 128), jnp.float32),
            pltpu.VMEM((ZCHUNK,), jnp.float32),
            pltpu.VMEM_SHARED((SP_WORDS,), jnp.float32),
        ],
    )


def _gmm_body(a_hbm, pk_hbm, out_hbm, a_v, pk_v, out_v):
    c = lax.axis_index("c")
    s = lax.axis_index("s")
    wid = s * NC + c
    grp = wid % NGRP
    sb = wid // NGRP

    pltpu.sync_copy(a_hbm.at[grp], a_v)

    def _chunk(k, carry):
        off = sb * 1024 + k * 256
        pltpu.sync_copy(pk_hbm.at[pl.ds(off, 256), :], pk_v)

        def _sample(i, carry2):
            pks = [pk_v[i, pl.ds(u * 16, 16)] for u in range(4)]
            accs = [jnp.zeros((16,), jnp.float32) for _ in range(8)]
            for j in range(S):
                w = pks[j // 16][j % 16]
                row = a_v[pl.ds((w & 0xFFF) * 16, 16)]
                simv = plsc.bitcast(
                    jnp.full((16,), w & -4096, jnp.int32), jnp.float32)
                accs[j % 8] = accs[j % 8] + simv * row
            accs = [accs[2*t] + accs[2*t+1] for t in range(4)]
            accs = [accs[0] + accs[1], accs[2] + accs[3]]
            out_v[pl.ds(i * 16, 16)] = accs[0] + accs[1]
            return carry2

        lax.fori_loop(0, 256, _sample, 0)
        pltpu.sync_copy(out_v, out_hbm.at[grp, pl.ds(off * 16, 4096)])
        return carry

    lax.fori_loop(0, 4, _chunk, 0)


@functools.cache
def _gmm_call():
    return pl.kernel(
        _gmm_body,
        out_type=jax.ShapeDtypeStruct((NGRP, B * 16), jnp.float32),
        mesh=plsc.VectorSubcoreMesh(core_axis_name="c", subcore_axis_name="s",
                                    num_cores=NC, num_subcores=NSUB),
        compiler_params=pltpu.CompilerParams(needs_layout_passes=False),
        scratch_types=[
            pltpu.VMEM((NS * 16,), jnp.float32),
            pltpu.VMEM((256, S), jnp.int32),
            pltpu.VMEM((256 * 16,), jnp.float32),
        ],
    )


def kernel(grad_output_batch, sign_ids, similarities, edge_src, edge_dst,
           edge_weight, num_diseases):
    eta_g = _norm_call(grad_output_batch).reshape(B)

    src = edge_src.astype(jnp.int32)
    dst = edge_dst.astype(jnp.int32)
    w = edge_weight.astype(jnp.float32)
    npad = EPAD - src.shape[0]
    src = jnp.concatenate([src, jnp.zeros((npad,), jnp.int32)])
    dst = jnp.concatenate([dst, jnp.zeros((npad,), jnp.int32)])
    w = jnp.concatenate([w, jnp.zeros((npad,), jnp.float32)])
    nd_arr = jnp.full((16,), num_diseases, jnp.int32)

    a_grouped = _abuild_call()(src, dst, w, nd_arr)

    scaled = similarities * eta_g[:, None]
    sim_bits = jax.lax.bitcast_convert_type(scaled, jnp.int32)
    packed = (sim_bits & -4096) | sign_ids.astype(jnp.int32)

    out_g = _gmm_call()(a_grouped, packed)
    return out_g.reshape(NGRP, B, 16).transpose(1, 0, 2).reshape(B, ND)


# speedup vs baseline: 4.8516x; 1.0642x over previous
import functools

import jax
import jax.numpy as jnp
from jax import lax
from jax.experimental import pallas as pl
from jax.experimental.pallas import tpu as pltpu
from jax.experimental.pallas import tpu_sc as plsc

ETA = 0.01
B, F, S, NS, ND = 4096, 2048, 64, 4096, 128
NC, NSUB, LANES = 2, 16, 16
NGRP = ND // 16
GRP_WORDS = NS * 16
PAD_BASE = 4 * GRP_WORDS
SP_WORDS = PAD_BASE + 2048
ZCHUNK = SP_WORDS // NSUB

EPT = 3328
EPAD = EPT * NSUB
NCH = EPT // 128


def _norm_body(x_ref, o_ref):
    x = x_ref[...]
    o_ref[...] = ETA * jnp.sqrt(jnp.sum(x * x, axis=1, keepdims=True))


_norm_call = pl.pallas_call(
    _norm_body,
    grid=(8,),
    in_specs=[pl.BlockSpec((B // 8, F), lambda i: (i, 0))],
    out_specs=pl.BlockSpec((B // 8, 1), lambda i: (i, 0)),
    out_shape=jax.ShapeDtypeStruct((B, 1), jnp.float32),
)


def _abuild_body(src_hbm, dst_hbm, w_hbm, nd_hbm, out_hbm,
                 src_v, dst_v, w_v, nd_v, idx_v, upd_v, zbuf, a_sp, sc_sem):
    c = lax.axis_index("c")
    s = lax.axis_index("s")

    zero = jnp.zeros((16,), jnp.float32)

    def _zb(i, carry):
        zbuf[pl.ds(i * 16, 16)] = zero
        return carry

    lax.fori_loop(0, ZCHUNK // 16, _zb, 0)
    pltpu.sync_copy(zbuf, a_sp.at[pl.ds(s * ZCHUNK, ZCHUNK)])
    plsc.subcore_barrier()

    base = s * EPT
    pltpu.sync_copy(src_hbm.at[pl.ds(base, EPT)], src_v)
    pltpu.sync_copy(dst_hbm.at[pl.ds(base, EPT)], dst_v)
    pltpu.sync_copy(w_hbm.at[pl.ds(base, EPT)], w_v)
    pltpu.sync_copy(nd_hbm, nd_v)

    lo = c * (ND // NC)
    ndv = nd_v[...]
    iota = lax.iota(jnp.int32, 16)
    pad_slots = PAD_BASE + iota * 16 + s

    def _chunk(r, carry):
        for k in range(8):
            off = r * 128 + k * 16
            sv = src_v[pl.ds(off, 16)]
            dv = dst_v[pl.ds(off, 16)]
            wv = w_v[pl.ds(off, 16)]
            local = sv - lo
            ok = (sv >= lo) & (sv < lo + (ND // NC)) & (sv < ndv)
            flat = (local >> 4) * GRP_WORDS + dv * 16 + (local & 15)
            idx_v[r, pl.ds(k * 16, 16)] = jnp.where(ok, flat, pad_slots)
            upd_v[r, pl.ds(k * 16, 16)] = jnp.where(ok, wv, 0.0)
        pltpu.async_copy(upd_v.at[r], a_sp.at[idx_v.at[r]], sc_sem, add=True)
        return carry

    lax.fori_loop(0, NCH, _chunk, 0)

    def _drain(r, carry):
        pltpu.make_async_copy(upd_v.at[r], a_sp.at[idx_v.at[r]], sc_sem).wait()
        return carry

    lax.fori_loop(0, NCH, _drain, 0)
    plsc.subcore_barrier()

    @pl.when(s < 4)
    def _():
        pltpu.sync_copy(a_sp.at[pl.ds(s * GRP_WORDS, GRP_WORDS)],
                        out_hbm.at[c * 4 + s])


@functools.cache
def _abuild_call():
    return pl.kernel(
        _abuild_body,
        out_type=jax.ShapeDtypeStruct((NGRP, GRP_WORDS), jnp.float32),
        mesh=plsc.VectorSubcoreMesh(core_axis_name="c", subcore_axis_name="s",
                                    num_cores=NC, num_subcores=NSUB),
        compiler_params=pltpu.CompilerParams(needs_layout_passes=False),
        scratch_types=[
            pltpu.VMEM((EPT,), jnp.int32),
            pltpu.VMEM((EPT,), jnp.int32),
            pltpu.VMEM((EPT,), jnp.float32),
            pltpu.VMEM((16,), jnp.int32),
            pltpu.VMEM((NCH, 128), jnp.int32),
            pltpu.VMEM((NCH, 128), jnp.float32),
            pltpu.VMEM((ZCHUNK,), jnp.float32),
            pltpu.VMEM_SHARED((SP_WORDS,), jnp.float32),
            pltpu.SemaphoreType.DMA,
        ],
    )


def _gmm_body(a_hbm, pk_hbm, out_hbm, a_v, pk_v, out_v):
    c = lax.axis_index("c")
    s = lax.axis_index("s")
    wid = s * NC + c
    grp = wid % NGRP
    sb = wid // NGRP

    pltpu.sync_copy(a_hbm.at[grp], a_v)

    def _chunk(k, carry):
        off = sb * 1024 + k * 256
        pltpu.sync_copy(pk_hbm.at[pl.ds(off, 256), :], pk_v)

        def _pair(m, carry2):
            i0 = m * 2
            pk0 = [pk_v[i0, pl.ds(u * 16, 16)] for u in range(4)]
            pk1 = [pk_v[i0 + 1, pl.ds(u * 16, 16)] for u in range(4)]
            a0 = [jnp.zeros((16,), jnp.float32) for _ in range(4)]
            a1 = [jnp.zeros((16,), jnp.float32) for _ in range(4)]
            for j in range(S):
                w0 = pk0[j // 16][j % 16]
                w1 = pk1[j // 16][j % 16]
                r0 = a_v[pl.ds((w0 & 0xFFF) * 16, 16)]
                r1 = a_v[pl.ds((w1 & 0xFFF) * 16, 16)]
                s0 = plsc.bitcast(jnp.full((16,), w0 & -4096, jnp.int32),
                                  jnp.float32)
                s1 = plsc.bitcast(jnp.full((16,), w1 & -4096, jnp.int32),
                                  jnp.float32)
                a0[j % 4] = a0[j % 4] + s0 * r0
                a1[j % 4] = a1[j % 4] + s1 * r1
            out_v[pl.ds(i0 * 16, 16)] = (a0[0] + a0[1]) + (a0[2] + a0[3])
            out_v[pl.ds(i0 * 16 + 16, 16)] = (a1[0] + a1[1]) + (a1[2] + a1[3])
            return carry2

        lax.fori_loop(0, 128, _pair, 0)
        pltpu.sync_copy(out_v, out_hbm.at[grp, pl.ds(off * 16, 4096)])
        return carry

    lax.fori_loop(0, 4, _chunk, 0)


@functools.cache
def _gmm_call():
    return pl.kernel(
        _gmm_body,
        out_type=jax.ShapeDtypeStruct((NGRP, B * 16), jnp.float32),
        mesh=plsc.VectorSubcoreMesh(core_axis_name="c", subcore_axis_name="s",
                                    num_cores=NC, num_subcores=NSUB),
        compiler_params=pltpu.CompilerParams(needs_layout_passes=False),
        scratch_types=[
            pltpu.VMEM((NS * 16,), jnp.float32),
            pltpu.VMEM((256, S), jnp.int32),
            pltpu.VMEM((256 * 16,), jnp.float32),
        ],
    )


def kernel(grad_output_batch, sign_ids, similarities, edge_src, edge_dst,
           edge_weight, num_diseases):
    eta_g = _norm_call(grad_output_batch).reshape(B)

    src = edge_src.astype(jnp.int32)
    dst = edge_dst.astype(jnp.int32)
    w = edge_weight.astype(jnp.float32)
    npad = EPAD - src.shape[0]
    src = jnp.concatenate([src, jnp.zeros((npad,), jnp.int32)])
    dst = jnp.concatenate([dst, jnp.zeros((npad,), jnp.int32)])
    w = jnp.concatenate([w, jnp.zeros((npad,), jnp.float32)])
    nd_arr = jnp.full((16,), num_diseases, jnp.int32)

    a_grouped = _abuild_call()(src, dst, w, nd_arr)

    scaled = similarities * eta_g[:, None]
    sim_bits = jax.lax.bitcast_convert_type(scaled, jnp.int32)
    packed = (sim_bits & -4096) | sign_ids.astype(jnp.int32)

    out_g = _gmm_call()(a_grouped, packed)
    return out_g.reshape(NGRP, B, 16).transpose(1, 0, 2).reshape(B, ND)
